# baseline (speedup 1.0000x reference)
import functools

import jax
import jax.numpy as jnp
import numpy as np
from jax import lax
from jax.experimental import pallas as pl
from jax.experimental.pallas import tpu as pltpu
from jax.experimental.pallas import tpu_sc as plsc

DIMS = 3
T = 8192
N_COLS = 543 * DIMS
LIPS = ([61, 185, 40, 39, 37, 0, 267, 269, 270, 409, 291]
        + [146, 91, 181, 84, 17, 314, 405, 321, 375, 291]
        + [78, 191, 80, 81, 82, 13, 312, 311, 310, 415, 308]
        + [78, 95, 88, 178, 87, 14, 317, 402, 318, 324, 308])

NW = 32
FRAMES_PER_W = T // NW
CHUNK = 32
N_CHUNKS = FRAMES_PER_W // CHUNK
CHUNK_W = CHUNK * N_COLS
L = 16
N_GROUPS = 24
N_SLOTS = N_GROUPS * L
HL_GROUPS = (9, 10, 11, 12)
HR_GROUPS = (20, 21, 22, 23)
P_W = 2 * N_SLOTS + L


def _build_slot_table():
    slots = []

    def seg(cols):
        p = list(cols)
        while len(p) % L:
            p.append(p[-1])
        slots.extend(p)

    seg([lm * DIMS + d for lm in LIPS for d in range(DIMS)])
    seg([lm * DIMS + d for lm in range(468, 489) for d in range(DIMS)])
    seg([lm * DIMS + d for lm in range(489, 522) for d in range(DIMS)])
    seg([lm * DIMS + d for lm in range(522, 543) for d in range(DIMS)])
    return np.asarray(slots, np.int32)


_IDX_COL = _build_slot_table()

SL_LIPS = (0, 129)
SL_HL = (144, 207)
SL_POSE = (208, 307)
SL_HR = (320, 383)
N_FEAT = 129 + 63 + 99 + 63

_sel = np.concatenate([np.arange(*SL_LIPS), np.arange(*SL_HL),
                       np.arange(*SL_POSE), np.arange(*SL_HR)])
_G_np = np.zeros((N_SLOTS, N_FEAT), np.float32)
_G_np[_sel, np.arange(N_FEAT)] = 1.0


def _sc_body(x_hbm, idx_hbm, out_hbm,
             buf_a, buf_b, idx_v, acc_v, sem_a, sem_b, sem_i):
    c = lax.axis_index("c")
    s = lax.axis_index("s")
    wid = s * 2 + c
    base = wid * FRAMES_PER_W * N_COLS

    pltpu.make_async_copy(idx_hbm, idx_v, sem_i).start()
    pltpu.make_async_copy(idx_hbm, idx_v, sem_i).wait()

    for g in range(2 * N_GROUPS + 1):
        acc_v[pl.ds(g * L, L)] = jnp.zeros((L,), jnp.float32)

    bufs = (buf_a, buf_b)
    sems = (sem_a, sem_b)
    pltpu.make_async_copy(x_hbm.at[pl.ds(base, CHUNK_W)], bufs[0],
                          sems[0]).start()

    n_hl = jnp.float32(0.0)
    n_hr = jnp.float32(0.0)
    for chunk in range(N_CHUNKS):
        buf = bufs[chunk % 2]
        pltpu.make_async_copy(
            x_hbm.at[pl.ds(base + chunk * CHUNK_W, CHUNK_W)], buf,
            sems[chunk % 2]).wait()
        if chunk + 1 < N_CHUNKS:
            pltpu.make_async_copy(
                x_hbm.at[pl.ds(base + (chunk + 1) * CHUNK_W, CHUNK_W)],
                bufs[(chunk + 1) % 2], sems[(chunk + 1) % 2]).start()

        def frame_body(f, carry):
            nhl, nhr = carry
            fbase = jnp.zeros((L,), jnp.int32) + f * N_COLS

            def gather(g):
                return plsc.load_gather(buf, [fbase + idx_v[pl.ds(g * L, L)]])

            hl_v = [gather(g) for g in HL_GROUPS]
            hr_v = [gather(g) for g in HR_GROUPS]
            hl_nan = (hl_v[0] != hl_v[0])
            for v in hl_v[1:]:
                hl_nan = jnp.logical_or(hl_nan, v != v)
            hr_nan = (hr_v[0] != hr_v[0])
            for v in hr_v[1:]:
                hr_nan = jnp.logical_or(hr_nan, v != v)
            hl_bad = jnp.any(hl_nan)
            hr_bad = jnp.any(hr_nan)
            hl_bad_v = jnp.broadcast_to(hl_bad, (L,))
            hr_bad_v = jnp.broadcast_to(hr_bad, (L,))

            for g in range(N_GROUPS):
                if g in HL_GROUPS:
                    v = jnp.where(hl_bad_v, 0.0, hl_v[g - HL_GROUPS[0]])
                elif g in HR_GROUPS:
                    v = jnp.where(hr_bad_v, 0.0, hr_v[g - HR_GROUPS[0]])
                else:
                    v = gather(g)
                acc_v[pl.ds(g * L, L)] = acc_v[pl.ds(g * L, L)] + v
                acc_v[pl.ds((N_GROUPS + g) * L, L)] = (
                    acc_v[pl.ds((N_GROUPS + g) * L, L)] + v * v)

            nhl = nhl + 1.0 - hl_bad.astype(jnp.float32)
            nhr = nhr + 1.0 - hr_bad.astype(jnp.float32)
            return (nhl, nhr)

        n_hl, n_hr = lax.fori_loop(0, CHUNK, frame_body, (n_hl, n_hr))

    lane = lax.iota(jnp.int32, L)
    cnt = ((lane == 0).astype(jnp.float32) * n_hl
           + (lane == 1).astype(jnp.float32) * n_hr)
    acc_v[pl.ds(2 * N_SLOTS, L)] = cnt
    pltpu.make_async_copy(acc_v, out_hbm.at[wid], sem_i).start()
    pltpu.make_async_copy(acc_v, out_hbm.at[wid], sem_i).wait()


def _sc_partials(x_flat, idx):
    mesh = plsc.VectorSubcoreMesh(core_axis_name="c", subcore_axis_name="s")
    fn = functools.partial(
        pl.kernel, mesh=mesh,
        compiler_params=pltpu.CompilerParams(needs_layout_passes=False),
        out_type=jax.ShapeDtypeStruct((NW, P_W), jnp.float32),
        scratch_types=[
            pltpu.VMEM((CHUNK_W,), jnp.float32),
            pltpu.VMEM((CHUNK_W,), jnp.float32),
            pltpu.VMEM((N_SLOTS,), jnp.int32),
            pltpu.VMEM((P_W,), jnp.float32),
            pltpu.SemaphoreType.DMA,
            pltpu.SemaphoreType.DMA,
            pltpu.SemaphoreType.DMA,
        ],
    )(_sc_body)
    return fn(x_flat, idx)


def _finalize_kernel(p_ref, g_ref, out_ref):
    blk = p_ref[...]
    psum = jnp.sum(blk, axis=0, keepdims=True)
    s = psum[:, 0:N_SLOTS]
    q = psum[:, N_SLOTS:2 * N_SLOTS]
    n_hl = jnp.broadcast_to(psum[:, 2 * N_SLOTS:2 * N_SLOTS + 1],
                            (1, N_SLOTS))
    n_hr = jnp.broadcast_to(psum[:, 2 * N_SLOTS + 1:2 * N_SLOTS + 2],
                            (1, N_SLOTS))
    colv = lax.broadcasted_iota(jnp.int32, (1, N_SLOTS), 1)
    in_hl = jnp.logical_and(colv >= 144, colv < 208).astype(jnp.float32)
    in_hr = (colv >= 320).astype(jnp.float32)
    n = (jnp.float32(T) + (n_hl - T) * in_hl + (n_hr - T) * in_hr)
    mean = s / n
    var = (q - n * mean * mean) / (n - 1.0)
    std = jnp.sqrt(var)
    mean = jnp.where(jnp.isnan(mean), 0.0, mean)
    std = jnp.where(jnp.isnan(std), 0.0, std)
    g = g_ref[...]
    out_ref[0:1, :] = jnp.dot(mean, g, preferred_element_type=jnp.float32,
                              precision=jax.lax.Precision.HIGHEST)
    out_ref[1:2, :] = jnp.dot(std, g, preferred_element_type=jnp.float32,
                              precision=jax.lax.Precision.HIGHEST)


@jax.jit
def kernel(x):
    idx = jnp.asarray(_IDX_COL)
    partials = _sc_partials(x.reshape(-1), idx)
    g = jnp.asarray(_G_np)
    out = pl.pallas_call(
        _finalize_kernel,
        in_specs=[
            pl.BlockSpec((NW, P_W), lambda: (0, 0)),
            pl.BlockSpec((N_SLOTS, N_FEAT), lambda: (0, 0)),
        ],
        out_specs=pl.BlockSpec((2, N_FEAT), lambda: (0, 0)),
        out_shape=jax.ShapeDtypeStruct((2, N_FEAT), jnp.float32),
    )(partials, g)
    return out.reshape(2 * N_FEAT)


# speedup vs baseline: 595.0949x; 595.0949x over previous
import jax
import jax.numpy as jnp
import numpy as np
from jax import lax
from jax.experimental import pallas as pl
from jax.experimental.pallas import tpu as pltpu

DIMS = 3
T = 8192
N_LM = 543
LIPS = ([61, 185, 40, 39, 37, 0, 267, 269, 270, 409, 291]
        + [146, 91, 181, 84, 17, 314, 405, 321, 375, 291]
        + [78, 191, 80, 81, 82, 13, 312, 311, 310, 415, 308]
        + [78, 95, 88, 178, 87, 14, 317, 402, 318, 324, 308])
HL_LO, HL_HI = 468, 489
POSE_LO, POSE_HI = 489, 522
HR_LO, HR_HI = 522, 543

_FEAT_LM = np.asarray(
    LIPS + list(range(HL_LO, HL_HI)) + list(range(POSE_LO, POSE_HI))
    + list(range(HR_LO, HR_HI)), np.int32)
N_FD = _FEAT_LM.shape[0]
_G_np = np.zeros((N_LM, N_FD), np.float32)
_G_np[_FEAT_LM, np.arange(N_FD)] = 1.0

FB = 1024


def _stats_kernel(x_ref, g_ref, out_ref, acc_sum, acc_ssq, acc_n):
    i = pl.program_id(0)
    nsteps = pl.num_programs(0)

    @pl.when(i == 0)
    def _init():
        acc_sum[...] = jnp.zeros_like(acc_sum)
        acc_ssq[...] = jnp.zeros_like(acc_ssq)
        acc_n[0] = 0.0
        acc_n[1] = 0.0

    blk = x_ref[...]
    hl = blk[:, HL_LO:HL_HI, :]
    hr = blk[:, HR_LO:HR_HI, :]
    hl_bad = jnp.any(jnp.any(jnp.isnan(hl), axis=1, keepdims=True),
                     axis=0, keepdims=True).astype(jnp.float32)
    hr_bad = jnp.any(jnp.any(jnp.isnan(hr), axis=1, keepdims=True),
                     axis=0, keepdims=True).astype(jnp.float32)

    lm = lax.broadcasted_iota(jnp.int32, (1, N_LM, 1), 1)
    is_hl = jnp.logical_and(lm >= HL_LO, lm < HL_HI).astype(jnp.float32)
    is_hr = (lm >= HR_LO).astype(jnp.float32)
    wbad = jnp.broadcast_to(hl_bad * is_hl + hr_bad * is_hr,
                            (DIMS, N_LM, FB))
    blk2 = jnp.where(wbad == 0.0, blk, 0.0)

    acc_sum[...] += jnp.sum(blk2, axis=2)
    acc_ssq[...] += jnp.sum(blk2 * blk2, axis=2)
    acc_n[0] += jnp.float32(FB) - jnp.sum(hl_bad)
    acc_n[1] += jnp.float32(FB) - jnp.sum(hr_bad)

    @pl.when(i == nsteps - 1)
    def _finalize():
        s = acc_sum[...]
        q = acc_ssq[...]
        lm2 = lax.broadcasted_iota(jnp.int32, (DIMS, N_LM), 1)
        in_hl = jnp.logical_and(lm2 >= HL_LO, lm2 < HL_HI).astype(jnp.float32)
        in_hr = (lm2 >= HR_LO).astype(jnp.float32)
        n = (jnp.float32(T) + (acc_n[0] - T) * in_hl
             + (acc_n[1] - T) * in_hr)
        mean = s / n
        var = (q - n * mean * mean) / (n - 1.0)
        std = jnp.sqrt(var)
        mean = jnp.where(jnp.isnan(mean), 0.0, mean)
        std = jnp.where(jnp.isnan(std), 0.0, std)
        g = g_ref[...]
        for d in range(DIMS):
            out_ref[d:d + 1, :] = jnp.dot(
                mean[d:d + 1, :], g, preferred_element_type=jnp.float32,
                precision=jax.lax.Precision.HIGHEST)
            out_ref[DIMS + d:DIMS + d + 1, :] = jnp.dot(
                std[d:d + 1, :], g, preferred_element_type=jnp.float32,
                precision=jax.lax.Precision.HIGHEST)


@jax.jit
def kernel(x):
    xt = x.transpose(2, 1, 0)
    g = jnp.asarray(_G_np)
    out = pl.pallas_call(
        _stats_kernel,
        grid=(T // FB,),
        in_specs=[
            pl.BlockSpec((DIMS, N_LM, FB), lambda i: (0, 0, i)),
            pl.BlockSpec((N_LM, N_FD), lambda i: (0, 0)),
        ],
        out_specs=pl.BlockSpec((2 * DIMS, N_FD), lambda i: (0, 0)),
        out_shape=jax.ShapeDtypeStruct((2 * DIMS, N_FD), jnp.float32),
        scratch_shapes=[
            pltpu.VMEM((DIMS, N_LM), jnp.float32),
            pltpu.VMEM((DIMS, N_LM), jnp.float32),
            pltpu.SMEM((2,), jnp.float32),
        ],
    )(xt, g)
    mean_part = out[0:DIMS].T.reshape(DIMS * N_FD)
    std_part = out[DIMS:].T.reshape(DIMS * N_FD)
    return jnp.concatenate([mean_part, std_part])
